# baseline (speedup 1.0000x reference)
import functools

import jax
import jax.numpy as jnp
from jax import lax
from jax.experimental import pallas as pl
from jax.experimental.pallas import tpu as pltpu
from jax.experimental.pallas import tpu_sc as plsc

NC = 2
NS = 16
NW = NC * NS
LANES = 16

B, S, D = 4096, 200, 128
DB = D // LANES

TC_ROWS = 3584
SC_ROWS = B - TC_ROWS
RPW = SC_ROWS // NW if SC_ROWS else 0
NBUF = 4


def _sc_body(x_hbm, w_hbm, out_hbm, xbuf, wslab, oslab, *sems):
    cid = lax.axis_index("c")
    sid = lax.axis_index("s")
    wid = sid * NC + cid
    base = TC_ROWS + wid * RPW

    pltpu.sync_copy(w_hbm.at[pl.ds(base * S, RPW * S)], wslab)

    for j in range(NBUF):
        pltpu.async_copy(x_hbm.at[base + j], xbuf.at[j], sems[j])

    def bcast(wv, lane):
        idx = jnp.full((LANES, 1), lane, jnp.int32)
        dnums = lax.GatherDimensionNumbers(
            offset_dims=(), collapsed_slice_dims=(0,), start_index_map=(0,))
        return lax.gather(wv, idx, dnums, slice_sizes=(1,),
                          mode=lax.GatherScatterMode.PROMISE_IN_BOUNDS)

    def row_compute(row, j):
        rowbase = row * S
        nfull = S // LANES
        tail = S - nfull * LANES

        def cbody(c, carry):
            accs = list(carry[:DB])
            den = carry[DB]
            wv = wslab[pl.ds(rowbase + c * LANES, LANES)]
            for jj in range(LANES):
                s = c * LANES + jj
                wb = bcast(wv, jj)
                for db in range(DB):
                    accs[db] = accs[db] + xbuf[j, s, pl.ds(db * LANES, LANES)] * wb
                den = den + wb
            return tuple(accs) + (den,)

        init = tuple(jnp.zeros((LANES,), jnp.float32) for _ in range(DB + 1))
        res = lax.fori_loop(0, nfull, cbody, init)

        accs = list(res[:DB])
        den = res[DB]
        wv = wslab[pl.ds(rowbase + S - LANES, LANES)]
        for jj in range(tail):
            s = nfull * LANES + jj
            wb = bcast(wv, LANES - tail + jj)
            for db in range(DB):
                accs[db] = accs[db] + xbuf[j, s, pl.ds(db * LANES, LANES)] * wb
            den = den + wb

        inv = 1.0 / den
        for db in range(DB):
            oslab[row, pl.ds(db * LANES, LANES)] = accs[db] * inv

    def grp(g, carry):
        for j in range(NBUF):
            row = g * NBUF + j
            pltpu.make_async_copy(x_hbm.at[base + row], xbuf.at[j], sems[j]).wait()
            row_compute(row, j)
            nxt = row + NBUF

            @pl.when(nxt < RPW)
            def _():
                pltpu.async_copy(x_hbm.at[base + nxt], xbuf.at[j], sems[j])
        return carry

    lax.fori_loop(0, RPW // NBUF, grp, 0)

    pltpu.sync_copy(oslab, out_hbm.at[pl.ds(wid * RPW, RPW)])


def _run_sc(x, w_flat):
    mesh = plsc.VectorSubcoreMesh(
        core_axis_name="c", subcore_axis_name="s", num_cores=NC, num_subcores=NS
    )
    return pl.kernel(
        _sc_body,
        out_type=jax.ShapeDtypeStruct((SC_ROWS, D), jnp.float32),
        mesh=mesh,
        scratch_types=[
            pltpu.VMEM((NBUF, S, D), jnp.float32),
            pltpu.VMEM((RPW * S,), jnp.float32),
            pltpu.VMEM((RPW, D), jnp.float32),
        ] + [pltpu.SemaphoreType.DMA] * NBUF,
    )(x, w_flat)


TC_BLK = 128


def _tc_body(x_ref, w_ref, o_ref):
    x = x_ref[...]
    w = w_ref[...]
    num = jnp.sum(x * w[:, :, None], axis=1)
    den = jnp.sum(w, axis=1)
    o_ref[...] = num / den[:, None]


def _run_tc(x, w):
    return pl.pallas_call(
        _tc_body,
        grid=(TC_ROWS // TC_BLK,),
        in_specs=[
            pl.BlockSpec((TC_BLK, S, D), lambda i: (i, 0, 0)),
            pl.BlockSpec((TC_BLK, S), lambda i: (i, 0)),
        ],
        out_specs=pl.BlockSpec((TC_BLK, D), lambda i: (i, 0)),
        out_shape=jax.ShapeDtypeStruct((TC_ROWS, D), jnp.float32),
    )(x, w)


@jax.jit
def _run(x, w):
    if SC_ROWS == 0:
        return _run_tc(x, w)
    out_sc = _run_sc(x, w.reshape(B * S))
    out_tc = _run_tc(x, w)
    return jnp.concatenate([out_tc, out_sc], axis=0)


def kernel(inputs, item_id_seq_weight):
    return _run(inputs, item_id_seq_weight.astype(jnp.float32))


# speedup vs baseline: 1.0511x; 1.0511x over previous
import functools

import jax
import jax.numpy as jnp
from jax import lax
from jax.experimental import pallas as pl
from jax.experimental.pallas import tpu as pltpu
from jax.experimental.pallas import tpu_sc as plsc

NC = 2
NS = 16
NW = NC * NS
LANES = 16

B, S, D = 4096, 200, 128
DB = D // LANES

TC_ROWS = 2304
SC_ROWS = B - TC_ROWS
RPW = SC_ROWS // NW if SC_ROWS else 0
NBUF = 4


def _sc_body(x_hbm, w_hbm, out_hbm, xbuf, wslab, oslab, *sems):
    cid = lax.axis_index("c")
    sid = lax.axis_index("s")
    wid = sid * NC + cid
    base = TC_ROWS + wid * RPW

    pltpu.sync_copy(w_hbm.at[pl.ds(base, RPW)], wslab)

    for j in range(NBUF):
        pltpu.async_copy(x_hbm.at[base + j], xbuf.at[j], sems[j])

    def bcast(wv, lane):
        idx = jnp.full((LANES, 1), lane, jnp.int32)
        dnums = lax.GatherDimensionNumbers(
            offset_dims=(), collapsed_slice_dims=(0,), start_index_map=(0,))
        return lax.gather(wv, idx, dnums, slice_sizes=(1,),
                          mode=lax.GatherScatterMode.PROMISE_IN_BOUNDS)

    def row_compute(row, j):

        nfull = S // LANES
        tail = S - nfull * LANES

        def cbody(c, carry):
            accs = list(carry[:DB])
            den = carry[DB]
            wv = wslab[row, pl.ds(c * LANES, LANES)]
            for jj in range(LANES):
                s = c * LANES + jj
                wb = bcast(wv, jj)
                for db in range(DB):
                    accs[db] = accs[db] + xbuf[j, s, pl.ds(db * LANES, LANES)] * wb
                den = den + wb
            return tuple(accs) + (den,)

        init = tuple(jnp.zeros((LANES,), jnp.float32) for _ in range(DB + 1))
        res = lax.fori_loop(0, nfull, cbody, init)

        accs = list(res[:DB])
        den = res[DB]
        wv = wslab[row, pl.ds(S - LANES, LANES)]
        for jj in range(tail):
            s = nfull * LANES + jj
            wb = bcast(wv, LANES - tail + jj)
            for db in range(DB):
                accs[db] = accs[db] + xbuf[j, s, pl.ds(db * LANES, LANES)] * wb
            den = den + wb

        inv = 1.0 / den
        for db in range(DB):
            oslab[row, pl.ds(db * LANES, LANES)] = accs[db] * inv

    def grp(g, carry):
        for j in range(NBUF):
            row = g * NBUF + j
            pltpu.make_async_copy(x_hbm.at[base + row], xbuf.at[j], sems[j]).wait()
            row_compute(row, j)
            nxt = row + NBUF

            @pl.when(nxt < RPW)
            def _():
                pltpu.async_copy(x_hbm.at[base + nxt], xbuf.at[j], sems[j])
        return carry

    lax.fori_loop(0, RPW // NBUF, grp, 0)

    pltpu.sync_copy(oslab, out_hbm.at[pl.ds(wid * RPW, RPW)])


def _run_sc(x, w):
    mesh = plsc.VectorSubcoreMesh(
        core_axis_name="c", subcore_axis_name="s", num_cores=NC, num_subcores=NS
    )
    return pl.kernel(
        _sc_body,
        out_type=jax.ShapeDtypeStruct((SC_ROWS, D), jnp.float32),
        mesh=mesh,
        scratch_types=[
            pltpu.VMEM((NBUF, S, D), jnp.float32),
            pltpu.VMEM((RPW, S), jnp.float32),
            pltpu.VMEM((RPW, D), jnp.float32),
        ] + [pltpu.SemaphoreType.DMA] * NBUF,
    )(x, w)


TC_BLK = 128


def _tc_body(x_ref, w_ref, o_ref):
    x = x_ref[...]
    w = w_ref[...]
    num = jnp.sum(x * w[:, :, None], axis=1)
    den = jnp.sum(w, axis=1)
    o_ref[...] = num / den[:, None]


def _run_tc(x, w):
    return pl.pallas_call(
        _tc_body,
        grid=(TC_ROWS // TC_BLK,),
        in_specs=[
            pl.BlockSpec((TC_BLK, S, D), lambda i: (i, 0, 0)),
            pl.BlockSpec((TC_BLK, S), lambda i: (i, 0)),
        ],
        out_specs=pl.BlockSpec((TC_BLK, D), lambda i: (i, 0)),
        out_shape=jax.ShapeDtypeStruct((TC_ROWS, D), jnp.float32),
    )(x, w)


@jax.jit
def _run(x, w):
    if SC_ROWS == 0:
        return _run_tc(x, w)
    out_sc = _run_sc(x, w)
    out_tc = _run_tc(x, w)
    return jnp.concatenate([out_tc, out_sc], axis=0)


def kernel(inputs, item_id_seq_weight):
    return _run(inputs, item_id_seq_weight.astype(jnp.float32))


# speedup vs baseline: 1.0693x; 1.0173x over previous
import functools

import jax
import jax.numpy as jnp
from jax import lax
from jax.experimental import pallas as pl
from jax.experimental.pallas import tpu as pltpu
from jax.experimental.pallas import tpu_sc as plsc

NC = 2
NS = 16
NW = NC * NS
LANES = 16

B, S, D = 4096, 200, 128
DB = D // LANES

TC_ROWS = 2304
SC_ROWS = B - TC_ROWS
RPW = SC_ROWS // NW if SC_ROWS else 0
NBUF = 4


def _sc_body(x_hbm, w_hbm, out_hbm, xbuf, wslab, oslab, *sems):
    cid = lax.axis_index("c")
    sid = lax.axis_index("s")
    wid = sid * NC + cid
    base = TC_ROWS + wid * RPW

    pltpu.sync_copy(w_hbm.at[pl.ds(base, RPW)], wslab)

    for j in range(NBUF):
        pltpu.async_copy(x_hbm.at[base + j], xbuf.at[j], sems[j])

    def bcast(wv, lane):
        idx = jnp.full((LANES, 1), lane, jnp.int32)
        dnums = lax.GatherDimensionNumbers(
            offset_dims=(), collapsed_slice_dims=(0,), start_index_map=(0,))
        return lax.gather(wv, idx, dnums, slice_sizes=(1,),
                          mode=lax.GatherScatterMode.PROMISE_IN_BOUNDS)

    def row_compute(row, j):

        nfull = S // LANES
        tail = S - nfull * LANES

        def cbody(c, carry):
            accs = list(carry[:DB])
            den = carry[DB]
            wv = wslab[row, pl.ds(c * LANES, LANES)]
            for jj in range(LANES):
                s = c * LANES + jj
                wb = bcast(wv, jj)
                for db in range(DB):
                    accs[db] = accs[db] + xbuf[j, s, pl.ds(db * LANES, LANES)] * wb
                den = den + wb
            return tuple(accs) + (den,)

        init = tuple(jnp.zeros((LANES,), jnp.float32) for _ in range(DB + 1))
        res = lax.fori_loop(0, nfull, cbody, init)

        accs = list(res[:DB])
        den = res[DB]
        wv = wslab[row, pl.ds(S - LANES, LANES)]
        for jj in range(tail):
            s = nfull * LANES + jj
            wb = bcast(wv, LANES - tail + jj)
            for db in range(DB):
                accs[db] = accs[db] + xbuf[j, s, pl.ds(db * LANES, LANES)] * wb
            den = den + wb

        inv = 1.0 / den
        for db in range(DB):
            oslab[row, pl.ds(db * LANES, LANES)] = accs[db] * inv

    def grp(g, carry):
        for j in range(NBUF):
            row = g * NBUF + j
            pltpu.make_async_copy(x_hbm.at[base + row], xbuf.at[j], sems[j]).wait()
            row_compute(row, j)
            nxt = row + NBUF

            @pl.when(nxt < RPW)
            def _():
                pltpu.async_copy(x_hbm.at[base + nxt], xbuf.at[j], sems[j])
        return carry

    lax.fori_loop(0, RPW // NBUF, grp, 0)

    pltpu.sync_copy(oslab, out_hbm.at[pl.ds(wid * RPW, RPW)])


def _run_sc(x, w):
    mesh = plsc.VectorSubcoreMesh(
        core_axis_name="c", subcore_axis_name="s", num_cores=NC, num_subcores=NS
    )
    return pl.kernel(
        _sc_body,
        out_type=jax.ShapeDtypeStruct((SC_ROWS, D), jnp.float32),
        mesh=mesh,
        scratch_types=[
            pltpu.VMEM((NBUF, S, D), jnp.float32),
            pltpu.VMEM((RPW, S), jnp.float32),
            pltpu.VMEM((RPW, D), jnp.float32),
        ] + [pltpu.SemaphoreType.DMA] * NBUF,
    )(x, w)


TC_BLK = 64
TC_NB = 4
TC_STEPS = TC_ROWS // TC_BLK


def _tc_body(x_hbm, w_hbm, o_ref, xbuf, wbuf, xsems, wsems):
    i = pl.program_id(0)

    def fetch(blk, slot):
        rows = pl.ds(blk * TC_BLK, TC_BLK)
        pltpu.make_async_copy(x_hbm.at[rows], xbuf.at[slot], xsems.at[slot]).start()
        pltpu.make_async_copy(w_hbm.at[rows], wbuf.at[slot], wsems.at[slot]).start()

    @pl.when(i == 0)
    def _():
        for b in range(TC_NB):
            fetch(b, b)

    slot = lax.rem(i, TC_NB)
    nxt = i + TC_NB

    pltpu.make_async_copy(x_hbm.at[pl.ds(0, TC_BLK)], xbuf.at[slot], xsems.at[slot]).wait()
    pltpu.make_async_copy(w_hbm.at[pl.ds(0, TC_BLK)], wbuf.at[slot], wsems.at[slot]).wait()

    x = xbuf[slot]
    w = wbuf[slot]
    num = jnp.sum(x * w[:, :, None], axis=1)
    den = jnp.sum(w, axis=1)
    o_ref[...] = num / den[:, None]

    @pl.when(nxt < TC_STEPS)
    def _():
        fetch(nxt, slot)


def _run_tc(x, w):
    return pl.pallas_call(
        _tc_body,
        grid=(TC_STEPS,),
        in_specs=[
            pl.BlockSpec(memory_space=pl.ANY),
            pl.BlockSpec(memory_space=pl.ANY),
        ],
        out_specs=pl.BlockSpec((TC_BLK, D), lambda i: (i, 0)),
        out_shape=jax.ShapeDtypeStruct((TC_ROWS, D), jnp.float32),
        scratch_shapes=[
            pltpu.VMEM((TC_NB, TC_BLK, S, D), jnp.float32),
            pltpu.VMEM((TC_NB, TC_BLK, S), jnp.float32),
            pltpu.SemaphoreType.DMA((TC_NB,)),
            pltpu.SemaphoreType.DMA((TC_NB,)),
        ],
    )(x, w)


@jax.jit
def _run(x, w):
    if SC_ROWS == 0:
        return _run_tc(x, w)
    out_sc = _run_sc(x, w)
    out_tc = _run_tc(x, w)
    return jnp.concatenate([out_tc, out_sc], axis=0)


def kernel(inputs, item_id_seq_weight):
    return _run(inputs, item_id_seq_weight.astype(jnp.float32))


# speedup vs baseline: 1.2533x; 1.1721x over previous
import functools

import jax
import jax.numpy as jnp
from jax import lax
from jax.experimental import pallas as pl
from jax.experimental.pallas import tpu as pltpu
from jax.experimental.pallas import tpu_sc as plsc

NC = 2
NS = 16
NW = NC * NS
LANES = 16

B, S, D = 4096, 200, 128
DB = D // LANES

TC_ROWS = 4096
SC_ROWS = B - TC_ROWS
RPW = SC_ROWS // NW if SC_ROWS else 0
NBUF = 4


def _sc_body(x_hbm, w_hbm, out_hbm, xbuf, wslab, oslab, *sems):
    cid = lax.axis_index("c")
    sid = lax.axis_index("s")
    wid = sid * NC + cid
    base = TC_ROWS + wid * RPW

    pltpu.sync_copy(w_hbm.at[pl.ds(base, RPW)], wslab)

    for j in range(NBUF):
        pltpu.async_copy(x_hbm.at[base + j], xbuf.at[j], sems[j])

    def bcast(wv, lane):
        idx = jnp.full((LANES, 1), lane, jnp.int32)
        dnums = lax.GatherDimensionNumbers(
            offset_dims=(), collapsed_slice_dims=(0,), start_index_map=(0,))
        return lax.gather(wv, idx, dnums, slice_sizes=(1,),
                          mode=lax.GatherScatterMode.PROMISE_IN_BOUNDS)

    def row_compute(row, j):

        nfull = S // LANES
        tail = S - nfull * LANES

        def cbody(c, carry):
            accs = list(carry[:DB])
            den = carry[DB]
            wv = wslab[row, pl.ds(c * LANES, LANES)]
            for jj in range(LANES):
                s = c * LANES + jj
                wb = bcast(wv, jj)
                for db in range(DB):
                    accs[db] = accs[db] + xbuf[j, s, pl.ds(db * LANES, LANES)] * wb
                den = den + wb
            return tuple(accs) + (den,)

        init = tuple(jnp.zeros((LANES,), jnp.float32) for _ in range(DB + 1))
        res = lax.fori_loop(0, nfull, cbody, init)

        accs = list(res[:DB])
        den = res[DB]
        wv = wslab[row, pl.ds(S - LANES, LANES)]
        for jj in range(tail):
            s = nfull * LANES + jj
            wb = bcast(wv, LANES - tail + jj)
            for db in range(DB):
                accs[db] = accs[db] + xbuf[j, s, pl.ds(db * LANES, LANES)] * wb
            den = den + wb

        inv = 1.0 / den
        for db in range(DB):
            oslab[row, pl.ds(db * LANES, LANES)] = accs[db] * inv

    def grp(g, carry):
        for j in range(NBUF):
            row = g * NBUF + j
            pltpu.make_async_copy(x_hbm.at[base + row], xbuf.at[j], sems[j]).wait()
            row_compute(row, j)
            nxt = row + NBUF

            @pl.when(nxt < RPW)
            def _():
                pltpu.async_copy(x_hbm.at[base + nxt], xbuf.at[j], sems[j])
        return carry

    lax.fori_loop(0, RPW // NBUF, grp, 0)

    pltpu.sync_copy(oslab, out_hbm.at[pl.ds(wid * RPW, RPW)])


def _run_sc(x, w):
    mesh = plsc.VectorSubcoreMesh(
        core_axis_name="c", subcore_axis_name="s", num_cores=NC, num_subcores=NS
    )
    return pl.kernel(
        _sc_body,
        out_type=jax.ShapeDtypeStruct((SC_ROWS, D), jnp.float32),
        mesh=mesh,
        scratch_types=[
            pltpu.VMEM((NBUF, S, D), jnp.float32),
            pltpu.VMEM((RPW, S), jnp.float32),
            pltpu.VMEM((RPW, D), jnp.float32),
        ] + [pltpu.SemaphoreType.DMA] * NBUF,
    )(x, w)


TC_BLK = 64
TC_NB = 4
TC_STEPS = TC_ROWS // TC_BLK


def _tc_body(x_hbm, w_hbm, o_ref, xbuf, wbuf, xsems, wsems):
    i = pl.program_id(0)

    def fetch(blk, slot):
        rows = pl.ds(blk * TC_BLK, TC_BLK)
        pltpu.make_async_copy(x_hbm.at[rows], xbuf.at[slot], xsems.at[slot]).start()
        pltpu.make_async_copy(w_hbm.at[rows], wbuf.at[slot], wsems.at[slot]).start()

    @pl.when(i == 0)
    def _():
        for b in range(TC_NB):
            fetch(b, b)

    slot = lax.rem(i, TC_NB)
    nxt = i + TC_NB

    pltpu.make_async_copy(x_hbm.at[pl.ds(0, TC_BLK)], xbuf.at[slot], xsems.at[slot]).wait()
    pltpu.make_async_copy(w_hbm.at[pl.ds(0, TC_BLK)], wbuf.at[slot], wsems.at[slot]).wait()

    x = xbuf[slot]
    w = wbuf[slot]
    num = jnp.sum(x * w[:, :, None], axis=1)
    den = jnp.sum(w, axis=1)
    o_ref[...] = num / den[:, None]

    @pl.when(nxt < TC_STEPS)
    def _():
        fetch(nxt, slot)


def _run_tc(x, w):
    return pl.pallas_call(
        _tc_body,
        grid=(TC_STEPS,),
        in_specs=[
            pl.BlockSpec(memory_space=pl.ANY),
            pl.BlockSpec(memory_space=pl.ANY),
        ],
        out_specs=pl.BlockSpec((TC_BLK, D), lambda i: (i, 0)),
        out_shape=jax.ShapeDtypeStruct((TC_ROWS, D), jnp.float32),
        scratch_shapes=[
            pltpu.VMEM((TC_NB, TC_BLK, S, D), jnp.float32),
            pltpu.VMEM((TC_NB, TC_BLK, S), jnp.float32),
            pltpu.SemaphoreType.DMA((TC_NB,)),
            pltpu.SemaphoreType.DMA((TC_NB,)),
        ],
    )(x, w)


@jax.jit
def _run(x, w):
    if SC_ROWS == 0:
        return _run_tc(x, w)
    out_sc = _run_sc(x, w)
    out_tc = _run_tc(x, w)
    return jnp.concatenate([out_tc, out_sc], axis=0)


def kernel(inputs, item_id_seq_weight):
    return _run(inputs, item_id_seq_weight.astype(jnp.float32))


# speedup vs baseline: 1.2649x; 1.0093x over previous
import functools

import jax
import jax.numpy as jnp
from jax import lax
from jax.experimental import pallas as pl
from jax.experimental.pallas import tpu as pltpu
from jax.experimental.pallas import tpu_sc as plsc

NC = 2
NS = 16
NW = NC * NS
LANES = 16

B, S, D = 4096, 200, 128
DB = D // LANES

TC_ROWS = 4096
SC_ROWS = B - TC_ROWS
RPW = SC_ROWS // NW if SC_ROWS else 0
NBUF = 4


def _sc_body(x_hbm, w_hbm, out_hbm, xbuf, wslab, oslab, *sems):
    cid = lax.axis_index("c")
    sid = lax.axis_index("s")
    wid = sid * NC + cid
    base = TC_ROWS + wid * RPW

    pltpu.sync_copy(w_hbm.at[pl.ds(base, RPW)], wslab)

    for j in range(NBUF):
        pltpu.async_copy(x_hbm.at[base + j], xbuf.at[j], sems[j])

    def bcast(wv, lane):
        idx = jnp.full((LANES, 1), lane, jnp.int32)
        dnums = lax.GatherDimensionNumbers(
            offset_dims=(), collapsed_slice_dims=(0,), start_index_map=(0,))
        return lax.gather(wv, idx, dnums, slice_sizes=(1,),
                          mode=lax.GatherScatterMode.PROMISE_IN_BOUNDS)

    def row_compute(row, j):

        nfull = S // LANES
        tail = S - nfull * LANES

        def cbody(c, carry):
            accs = list(carry[:DB])
            den = carry[DB]
            wv = wslab[row, pl.ds(c * LANES, LANES)]
            for jj in range(LANES):
                s = c * LANES + jj
                wb = bcast(wv, jj)
                for db in range(DB):
                    accs[db] = accs[db] + xbuf[j, s, pl.ds(db * LANES, LANES)] * wb
                den = den + wb
            return tuple(accs) + (den,)

        init = tuple(jnp.zeros((LANES,), jnp.float32) for _ in range(DB + 1))
        res = lax.fori_loop(0, nfull, cbody, init)

        accs = list(res[:DB])
        den = res[DB]
        wv = wslab[row, pl.ds(S - LANES, LANES)]
        for jj in range(tail):
            s = nfull * LANES + jj
            wb = bcast(wv, LANES - tail + jj)
            for db in range(DB):
                accs[db] = accs[db] + xbuf[j, s, pl.ds(db * LANES, LANES)] * wb
            den = den + wb

        inv = 1.0 / den
        for db in range(DB):
            oslab[row, pl.ds(db * LANES, LANES)] = accs[db] * inv

    def grp(g, carry):
        for j in range(NBUF):
            row = g * NBUF + j
            pltpu.make_async_copy(x_hbm.at[base + row], xbuf.at[j], sems[j]).wait()
            row_compute(row, j)
            nxt = row + NBUF

            @pl.when(nxt < RPW)
            def _():
                pltpu.async_copy(x_hbm.at[base + nxt], xbuf.at[j], sems[j])
        return carry

    lax.fori_loop(0, RPW // NBUF, grp, 0)

    pltpu.sync_copy(oslab, out_hbm.at[pl.ds(wid * RPW, RPW)])


def _run_sc(x, w):
    mesh = plsc.VectorSubcoreMesh(
        core_axis_name="c", subcore_axis_name="s", num_cores=NC, num_subcores=NS
    )
    return pl.kernel(
        _sc_body,
        out_type=jax.ShapeDtypeStruct((SC_ROWS, D), jnp.float32),
        mesh=mesh,
        scratch_types=[
            pltpu.VMEM((NBUF, S, D), jnp.float32),
            pltpu.VMEM((RPW, S), jnp.float32),
            pltpu.VMEM((RPW, D), jnp.float32),
        ] + [pltpu.SemaphoreType.DMA] * NBUF,
    )(x, w)


TC_BLK = 64
TC_NB = 6
TC_STEPS = TC_ROWS // TC_BLK


def _tc_body(x_hbm, w_hbm, o_ref, xbuf, wbuf, xsems, wsems):
    i = pl.program_id(0)

    def fetch(blk, slot):
        rows = pl.ds(blk * TC_BLK, TC_BLK)
        pltpu.make_async_copy(x_hbm.at[rows], xbuf.at[slot], xsems.at[slot]).start()
        pltpu.make_async_copy(w_hbm.at[rows], wbuf.at[slot], wsems.at[slot]).start()

    @pl.when(i == 0)
    def _():
        for b in range(TC_NB):
            fetch(b, b)

    slot = lax.rem(i, TC_NB)
    nxt = i + TC_NB

    pltpu.make_async_copy(x_hbm.at[pl.ds(0, TC_BLK)], xbuf.at[slot], xsems.at[slot]).wait()
    pltpu.make_async_copy(w_hbm.at[pl.ds(0, TC_BLK)], wbuf.at[slot], wsems.at[slot]).wait()

    x = xbuf[slot]
    w = wbuf[slot]
    num = jnp.sum(x * w[:, :, None], axis=1)
    den = jnp.sum(w, axis=1)
    o_ref[...] = num / den[:, None]

    @pl.when(nxt < TC_STEPS)
    def _():
        fetch(nxt, slot)


def _run_tc(x, w):
    return pl.pallas_call(
        _tc_body,
        grid=(TC_STEPS,),
        in_specs=[
            pl.BlockSpec(memory_space=pl.ANY),
            pl.BlockSpec(memory_space=pl.ANY),
        ],
        out_specs=pl.BlockSpec((TC_BLK, D), lambda i: (i, 0)),
        out_shape=jax.ShapeDtypeStruct((TC_ROWS, D), jnp.float32),
        scratch_shapes=[
            pltpu.VMEM((TC_NB, TC_BLK, S, D), jnp.float32),
            pltpu.VMEM((TC_NB, TC_BLK, S), jnp.float32),
            pltpu.SemaphoreType.DMA((TC_NB,)),
            pltpu.SemaphoreType.DMA((TC_NB,)),
        ],
    )(x, w)


@jax.jit
def _run(x, w):
    if SC_ROWS == 0:
        return _run_tc(x, w)
    out_sc = _run_sc(x, w)
    out_tc = _run_tc(x, w)
    return jnp.concatenate([out_tc, out_sc], axis=0)


def kernel(inputs, item_id_seq_weight):
    return _run(inputs, item_id_seq_weight.astype(jnp.float32))
